# Initial kernel scaffold; baseline (speedup 1.0000x reference)
import jax
import jax.numpy as jnp
from jax.experimental import pallas as pl


def kernel(x, edge_index, W1, b1, W2, b2, Wt, bt, Wfc, bfc):
    raise NotImplementedError("write your pallas kernel here")


# speedup vs baseline: 22.9315x; 22.9315x over previous
import jax
import jax.numpy as jnp
from jax import lax
from jax.experimental import pallas as pl
from jax.experimental.pallas import tpu as pltpu
from jax.experimental.pallas import tpu_sc as plsc

N = 10000
D = 128
E = 320000
NC = 2
NS = 16
CH = 80
NCHUNK = 125
NPAD = 10240
RPT = NPAD // NS
ZROWS = 128
F32 = jnp.float32


CH_D = 80
NCHUNK_D = 125


def _deg_body(dst_hbm, out_hbm, dacc, idx_v, ones_v, zb_v, dsem):
    c = lax.axis_index("c")
    s = lax.axis_index("s")

    def fill(r, carry):
        ones_v[r, :] = jnp.full((16,), 1.0, F32)
        return carry
    lax.fori_loop(0, CH_D, fill, 0)

    def fillz(r, carry):
        zb_v[r, :] = jnp.zeros((16,), F32)
        return carry
    lax.fori_loop(0, ZROWS, fillz, 0)

    for k in range(RPT // ZROWS):
        pltpu.sync_copy(zb_v, dacc.at[pl.ds(s * RPT + k * ZROWS, ZROWS)])
    pltpu.sync_copy(dst_hbm.at[c, s], idx_v)
    plsc.subcore_barrier()

    def chunk(j, carry):
        pltpu.async_copy(ones_v, dacc.at[idx_v.at[j]], dsem, add=True)

        @pl.when(j >= 5)
        def _():
            pltpu.make_async_copy(ones_v, dacc.at[idx_v.at[0]], dsem).wait()
        return carry
    lax.fori_loop(0, NCHUNK_D, chunk, 0)
    for _ in range(5):
        pltpu.make_async_copy(ones_v, dacc.at[idx_v.at[0]], dsem).wait()

    plsc.subcore_barrier()
    pltpu.sync_copy(dacc.at[pl.ds(s * RPT, RPT)],
                    out_hbm.at[c, pl.ds(s * RPT, RPT)])


def _sc_deg(dst4):
    return pl.kernel(
        _deg_body,
        out_type=jax.ShapeDtypeStruct((NC, NPAD, 16), F32),
        mesh=plsc.VectorSubcoreMesh(core_axis_name="c", subcore_axis_name="s"),
        scratch_types=[
            pltpu.VMEM_SHARED((NPAD, 16), F32),
            pltpu.VMEM((NCHUNK_D, CH_D), jnp.int32),
            pltpu.VMEM((CH_D, 16), F32),
            pltpu.VMEM((ZROWS, 16), F32),
            pltpu.SemaphoreType.DMA,
        ],
    )(dst4)


def _scatter_body(z_hbm, src_hbm, dst_hbm, out_hbm, acc,
                  isv, idv, rows0, rows1, sem0, sem1, sema0, sema1):
    c = lax.axis_index("c")
    s = lax.axis_index("s")

    def fillz(r, carry):
        for k8 in range(D // 16):
            rows0[r, pl.ds(k8 * 16, 16)] = jnp.zeros((16,), F32)
        return carry
    lax.fori_loop(0, CH, fillz, 0)

    for k in range(RPT // CH):
        pltpu.sync_copy(rows0, acc.at[pl.ds(s * RPT + k * CH, CH)])
    pltpu.sync_copy(src_hbm.at[c, s], isv)
    pltpu.sync_copy(dst_hbm.at[c, s], idv)
    plsc.subcore_barrier()

    def gath(j, buf, sem):
        pltpu.async_copy(z_hbm.at[isv.at[pl.ds(j * CH, CH)]], buf, sem)

    def gwait(buf, sem):
        pltpu.make_async_copy(z_hbm.at[isv.at[pl.ds(0, CH)]], buf, sem).wait()

    def awaitp(buf, sem):
        pltpu.make_async_copy(buf, acc.at[idv.at[0]], sem).wait()

    gath(0, rows0, sem0)
    gath(1, rows1, sem1)

    def pair(t, carry):
        j = 2 * t
        gwait(rows0, sem0)
        pltpu.async_copy(rows0, acc.at[idv.at[j]], sema0, add=True)
        gwait(rows1, sem1)
        pltpu.async_copy(rows1, acc.at[idv.at[j + 1]], sema1, add=True)
        awaitp(rows0, sema0)
        gath(j + 2, rows0, sem0)

        @pl.when(t < (NCHUNK - 1) // 2 - 1)
        def _():
            awaitp(rows1, sema1)
            gath(j + 3, rows1, sem1)
        return carry
    lax.fori_loop(0, (NCHUNK - 1) // 2, pair, 0)

    last = NCHUNK - 1
    gwait(rows0, sem0)
    pltpu.async_copy(rows0, acc.at[idv.at[last]], sema0, add=True)
    awaitp(rows0, sema0)
    awaitp(rows1, sema1)

    plsc.subcore_barrier()
    pltpu.sync_copy(acc.at[pl.ds(s * RPT, RPT)],
                    out_hbm.at[c, pl.ds(s * RPT, RPT)])


def _sc_scatter(z, src3, dst4):
    return pl.kernel(
        _scatter_body,
        out_type=jax.ShapeDtypeStruct((NC, NPAD, D), F32),
        mesh=plsc.VectorSubcoreMesh(core_axis_name="c", subcore_axis_name="s"),
        scratch_types=[
            pltpu.VMEM_SHARED((NPAD, D), F32),
            pltpu.VMEM((NCHUNK * CH,), jnp.int32),
            pltpu.VMEM((NCHUNK, CH), jnp.int32),
            pltpu.VMEM((CH, D), F32),
            pltpu.VMEM((CH, D), F32),
            pltpu.SemaphoreType.DMA,
            pltpu.SemaphoreType.DMA,
            pltpu.SemaphoreType.DMA,
            pltpu.SemaphoreType.DMA,
        ],
    )(z, src3, dst4)


_BLK = 1000
_GRID = N // _BLK
_PREC = lax.Precision.HIGHEST


def _prep_body(degp_ref, x_ref, w_ref, dinv_ref, z_ref):
    deg = degp_ref[0] + degp_ref[1]
    dinv = lax.rsqrt(deg[:, 0:1] + 1.0)
    dinv_ref[...] = dinv
    z_ref[...] = jnp.dot(x_ref[...], w_ref[...],
                         preferred_element_type=F32, precision=_PREC) * dinv


def _tc_prep(degp, x, W1):
    return pl.pallas_call(
        _prep_body,
        grid=(_GRID,),
        in_specs=[
            pl.BlockSpec((NC, _BLK, 16), lambda i: (0, i, 0)),
            pl.BlockSpec((_BLK, D), lambda i: (i, 0)),
            pl.BlockSpec((D, D), lambda i: (0, 0)),
        ],
        out_specs=[
            pl.BlockSpec((_BLK, 1), lambda i: (i, 0)),
            pl.BlockSpec((_BLK, D), lambda i: (i, 0)),
        ],
        out_shape=[
            jax.ShapeDtypeStruct((N, 1), F32),
            jax.ShapeDtypeStruct((N, D), F32),
        ],
    )(degp, x, W1)


def _mid_body(aggp_ref, z_ref, dinv_ref, b_ref, w_ref, out_ref):
    dinv = dinv_ref[...]
    h = (aggp_ref[0] + aggp_ref[1] + z_ref[...]) * dinv + b_ref[...]
    h = jnp.maximum(h, 0.0)
    out_ref[...] = jnp.dot(h, w_ref[...],
                           preferred_element_type=F32, precision=_PREC) * dinv


def _tc_mid(aggp, z, dinv, b1r, W2):
    return pl.pallas_call(
        _mid_body,
        grid=(_GRID,),
        in_specs=[
            pl.BlockSpec((NC, _BLK, D), lambda i: (0, i, 0)),
            pl.BlockSpec((_BLK, D), lambda i: (i, 0)),
            pl.BlockSpec((_BLK, 1), lambda i: (i, 0)),
            pl.BlockSpec((1, D), lambda i: (0, 0)),
            pl.BlockSpec((D, D), lambda i: (0, 0)),
        ],
        out_specs=pl.BlockSpec((_BLK, D), lambda i: (i, 0)),
        out_shape=jax.ShapeDtypeStruct((N, D), F32),
    )(aggp, z, dinv, b1r, W2)


def _act_body(aggp_ref, z_ref, dinv_ref, b_ref, h_ref):
    dinv = dinv_ref[...]
    h = (aggp_ref[0] + aggp_ref[1] + z_ref[...]) * dinv + b_ref[...]
    h_ref[...] = jnp.maximum(h, 0.0)


def _tc_act(aggp, z, dinv, b2r):
    return pl.pallas_call(
        _act_body,
        grid=(_GRID,),
        in_specs=[
            pl.BlockSpec((NC, _BLK, D), lambda i: (0, i, 0)),
            pl.BlockSpec((_BLK, D), lambda i: (i, 0)),
            pl.BlockSpec((_BLK, 1), lambda i: (i, 0)),
            pl.BlockSpec((1, D), lambda i: (0, 0)),
        ],
        out_specs=pl.BlockSpec((_BLK, D), lambda i: (i, 0)),
        out_shape=jax.ShapeDtypeStruct((N, D), F32),
    )(aggp, z, dinv, b2r)


def _tconv_body(hp_ref, hc_ref, hn_ref, wt_ref, btr_ref, wfc_ref, bfc_ref,
                out_ref):
    i = pl.program_id(0)
    hc = hc_ref[...]
    wfc = wfc_ref[...]
    dn = (((0,), (0,)), ((), ()))
    u0 = lax.dot_general(wt_ref[0], wfc, dn, precision=_PREC,
                         preferred_element_type=F32)
    u1 = lax.dot_general(wt_ref[1], wfc, dn, precision=_PREC,
                         preferred_element_type=F32)
    u2 = lax.dot_general(wt_ref[2], wfc, dn, precision=_PREC,
                         preferred_element_type=F32)
    cst = jnp.dot(btr_ref[...], wfc, preferred_element_type=F32,
                  precision=_PREC) + bfc_ref[...]
    zrow = jnp.zeros((1, D), F32)
    first = jnp.where(i > 0, hp_ref[7:8, :], zrow)
    lastr = jnp.where(i < _GRID - 1, hn_ref[0:1, :], zrow)
    prev = jnp.concatenate([first, hc[:-1]], axis=0)
    nxt = jnp.concatenate([hc[1:], lastr], axis=0)
    out = jnp.dot(prev, u0, preferred_element_type=F32, precision=_PREC)
    out += jnp.dot(hc, u1, preferred_element_type=F32, precision=_PREC)
    out += jnp.dot(nxt, u2, preferred_element_type=F32, precision=_PREC)
    out_ref[...] = out + cst


def _tc_tconv(h, wt_t, btr, Wfc, bfcr):
    nhalo = N // 8
    return pl.pallas_call(
        _tconv_body,
        grid=(_GRID,),
        in_specs=[
            pl.BlockSpec((8, D), lambda i: (jnp.maximum(i * (_BLK // 8) - 1, 0), 0)),
            pl.BlockSpec((_BLK, D), lambda i: (i, 0)),
            pl.BlockSpec((8, D), lambda i: (jnp.minimum((i + 1) * (_BLK // 8), nhalo - 1), 0)),
            pl.BlockSpec((3, D, D), lambda i: (0, 0, 0)),
            pl.BlockSpec((1, D), lambda i: (0, 0)),
            pl.BlockSpec((D, D), lambda i: (0, 0)),
            pl.BlockSpec((1, D), lambda i: (0, 0)),
        ],
        out_specs=pl.BlockSpec((_BLK, D), lambda i: (i, 0)),
        out_shape=jax.ShapeDtypeStruct((N, D), F32),
    )(h, h, h, wt_t, btr, Wfc, bfcr)


def kernel(x, edge_index, W1, b1, W2, b2, Wt, bt, Wfc, bfc):
    src3 = edge_index[0].reshape(NC, NS, NCHUNK * CH)
    dst4 = edge_index[1].reshape(NC, NS, NCHUNK, CH)
    degp = _sc_deg(dst4)
    dinv, z1 = _tc_prep(degp, x, W1)
    agg1 = _sc_scatter(z1, src3, dst4)
    z2 = _tc_mid(agg1, z1, dinv, b1.reshape(1, D), W2)
    agg2 = _sc_scatter(z2, src3, dst4)
    h2 = _tc_act(agg2, z2, dinv, b2.reshape(1, D))
    return _tc_tconv(h2, jnp.transpose(Wt, (2, 0, 1)), bt.reshape(1, D),
                     Wfc, bfc.reshape(1, D))


# speedup vs baseline: 26.5179x; 1.1564x over previous
import jax
import jax.numpy as jnp
from jax import lax
from jax.experimental import pallas as pl
from jax.experimental.pallas import tpu as pltpu
from jax.experimental.pallas import tpu_sc as plsc

N = 10000
D = 128
E = 320000
NC = 2
NS = 16
CH = 80
NCHUNK = 125
NPAD = 10240
RPT = NPAD // NS
ZROWS = 128
F32 = jnp.float32


CH_D = 80
NCHUNK_D = 125


def _deg_body(dst_hbm, out_hbm, dacc_a, dacc_b, idx_v, ones_v, zb_v,
              sema, semb):
    c = lax.axis_index("c")
    s = lax.axis_index("s")

    def fill(r, carry):
        ones_v[r, :] = jnp.full((16,), 1.0, F32)
        return carry
    lax.fori_loop(0, CH_D, fill, 0)

    def fillz(r, carry):
        zb_v[r, :] = jnp.zeros((16,), F32)
        return carry
    lax.fori_loop(0, ZROWS, fillz, 0)

    for k in range(RPT // ZROWS):
        pltpu.sync_copy(zb_v, dacc_a.at[pl.ds(s * RPT + k * ZROWS, ZROWS)])
        pltpu.sync_copy(zb_v, dacc_b.at[pl.ds(s * RPT + k * ZROWS, ZROWS)])
    pltpu.sync_copy(dst_hbm.at[c, s], idx_v)
    plsc.subcore_barrier()

    def adda(j):
        pltpu.async_copy(ones_v, dacc_a.at[idx_v.at[j]], sema, add=True)

    def addb(j):
        pltpu.async_copy(ones_v, dacc_b.at[idx_v.at[j]], semb, add=True)

    def waita():
        pltpu.make_async_copy(ones_v, dacc_a.at[idx_v.at[0]], sema).wait()

    def waitb():
        pltpu.make_async_copy(ones_v, dacc_b.at[idx_v.at[0]], semb).wait()

    adda(0)
    addb(1)

    def chunk(t, carry):
        j = 2 * t
        waita()
        adda(j + 2)
        waitb()
        addb(j + 3)
        return carry
    lax.fori_loop(0, (NCHUNK_D - 3) // 2, chunk, 0)
    waita()
    adda(NCHUNK_D - 1)
    waitb()
    waita()

    plsc.subcore_barrier()
    pltpu.sync_copy(dacc_a.at[pl.ds(s * RPT, RPT)],
                    out_hbm.at[c, 0, pl.ds(s * RPT, RPT)])
    pltpu.sync_copy(dacc_b.at[pl.ds(s * RPT, RPT)],
                    out_hbm.at[c, 1, pl.ds(s * RPT, RPT)])


def _sc_deg(dst4):
    return pl.kernel(
        _deg_body,
        out_type=jax.ShapeDtypeStruct((NC, 2, NPAD, 16), F32),
        mesh=plsc.VectorSubcoreMesh(core_axis_name="c", subcore_axis_name="s"),
        scratch_types=[
            pltpu.VMEM_SHARED((NPAD, 16), F32),
            pltpu.VMEM_SHARED((NPAD, 16), F32),
            pltpu.VMEM((NCHUNK_D, CH_D), jnp.int32),
            pltpu.VMEM((CH_D, 16), F32),
            pltpu.VMEM((ZROWS, 16), F32),
            pltpu.SemaphoreType.DMA,
            pltpu.SemaphoreType.DMA,
        ],
    )(dst4)


def _scatter_body(z_hbm, src_hbm, dst_hbm, out_hbm, acc,
                  isv, idv, rows0, rows1, sem0, sem1):
    c = lax.axis_index("c")
    s = lax.axis_index("s")

    def fillz(r, carry):
        for k8 in range(D // 16):
            rows0[r, pl.ds(k8 * 16, 16)] = jnp.zeros((16,), F32)
        return carry
    lax.fori_loop(0, CH, fillz, 0)

    for k in range(RPT // CH):
        pltpu.sync_copy(rows0, acc.at[pl.ds(s * RPT + k * CH, CH)])
    pltpu.sync_copy(src_hbm.at[c, s], isv)
    pltpu.sync_copy(dst_hbm.at[c, s], idv)
    plsc.subcore_barrier()

    pltpu.async_copy(z_hbm.at[isv.at[pl.ds(0, CH)]], rows0, sem0)

    def pair(t, carry):
        j = 2 * t
        pltpu.async_copy(z_hbm.at[isv.at[pl.ds((j + 1) * CH, CH)]], rows1, sem1)
        pltpu.make_async_copy(z_hbm.at[isv.at[pl.ds(j * CH, CH)]], rows0, sem0).wait()
        pltpu.sync_copy(rows0, acc.at[idv.at[j]], add=True)
        pltpu.async_copy(z_hbm.at[isv.at[pl.ds((j + 2) * CH, CH)]], rows0, sem0)
        pltpu.make_async_copy(z_hbm.at[isv.at[pl.ds((j + 1) * CH, CH)]], rows1, sem1).wait()
        pltpu.sync_copy(rows1, acc.at[idv.at[j + 1]], add=True)
        return carry
    lax.fori_loop(0, (NCHUNK - 1) // 2, pair, 0)

    last = NCHUNK - 1
    pltpu.make_async_copy(z_hbm.at[isv.at[pl.ds(last * CH, CH)]], rows0, sem0).wait()
    pltpu.sync_copy(rows0, acc.at[idv.at[last]], add=True)

    plsc.subcore_barrier()
    pltpu.sync_copy(acc.at[pl.ds(s * RPT, RPT)],
                    out_hbm.at[c, pl.ds(s * RPT, RPT)])


def _sc_scatter(z, src3, dst4):
    return pl.kernel(
        _scatter_body,
        out_type=jax.ShapeDtypeStruct((NC, NPAD, D), F32),
        mesh=plsc.VectorSubcoreMesh(core_axis_name="c", subcore_axis_name="s"),
        scratch_types=[
            pltpu.VMEM_SHARED((NPAD, D), F32),
            pltpu.VMEM((NCHUNK * CH,), jnp.int32),
            pltpu.VMEM((NCHUNK, CH), jnp.int32),
            pltpu.VMEM((CH, D), F32),
            pltpu.VMEM((CH, D), F32),
            pltpu.SemaphoreType.DMA,
            pltpu.SemaphoreType.DMA,
        ],
    )(z, src3, dst4)


_BLK = 1000
_GRID = N // _BLK
_PREC = lax.Precision.HIGHEST


def _prep_body(degp_ref, x_ref, w_ref, dinv_ref, z_ref):
    deg = (degp_ref[0, 0] + degp_ref[0, 1]
           + degp_ref[1, 0] + degp_ref[1, 1])
    dinv = lax.rsqrt(deg[:, 0:1] + 1.0)
    dinv_ref[...] = dinv
    z_ref[...] = jnp.dot(x_ref[...], w_ref[...],
                         preferred_element_type=F32, precision=_PREC) * dinv


def _tc_prep(degp, x, W1):
    return pl.pallas_call(
        _prep_body,
        grid=(_GRID,),
        in_specs=[
            pl.BlockSpec((NC, 2, _BLK, 16), lambda i: (0, 0, i, 0)),
            pl.BlockSpec((_BLK, D), lambda i: (i, 0)),
            pl.BlockSpec((D, D), lambda i: (0, 0)),
        ],
        out_specs=[
            pl.BlockSpec((_BLK, 1), lambda i: (i, 0)),
            pl.BlockSpec((_BLK, D), lambda i: (i, 0)),
        ],
        out_shape=[
            jax.ShapeDtypeStruct((N, 1), F32),
            jax.ShapeDtypeStruct((N, D), F32),
        ],
    )(degp, x, W1)


def _mid_body(aggp_ref, z_ref, dinv_ref, b_ref, w_ref, out_ref):
    dinv = dinv_ref[...]
    h = (aggp_ref[0] + aggp_ref[1] + z_ref[...]) * dinv + b_ref[...]
    h = jnp.maximum(h, 0.0)
    out_ref[...] = jnp.dot(h, w_ref[...],
                           preferred_element_type=F32, precision=_PREC) * dinv


def _tc_mid(aggp, z, dinv, b1r, W2):
    return pl.pallas_call(
        _mid_body,
        grid=(_GRID,),
        in_specs=[
            pl.BlockSpec((NC, _BLK, D), lambda i: (0, i, 0)),
            pl.BlockSpec((_BLK, D), lambda i: (i, 0)),
            pl.BlockSpec((_BLK, 1), lambda i: (i, 0)),
            pl.BlockSpec((1, D), lambda i: (0, 0)),
            pl.BlockSpec((D, D), lambda i: (0, 0)),
        ],
        out_specs=pl.BlockSpec((_BLK, D), lambda i: (i, 0)),
        out_shape=jax.ShapeDtypeStruct((N, D), F32),
    )(aggp, z, dinv, b1r, W2)


def _act_body(aggp_ref, z_ref, dinv_ref, b_ref, h_ref):
    dinv = dinv_ref[...]
    h = (aggp_ref[0] + aggp_ref[1] + z_ref[...]) * dinv + b_ref[...]
    h_ref[...] = jnp.maximum(h, 0.0)


def _tc_act(aggp, z, dinv, b2r):
    return pl.pallas_call(
        _act_body,
        grid=(_GRID,),
        in_specs=[
            pl.BlockSpec((NC, _BLK, D), lambda i: (0, i, 0)),
            pl.BlockSpec((_BLK, D), lambda i: (i, 0)),
            pl.BlockSpec((_BLK, 1), lambda i: (i, 0)),
            pl.BlockSpec((1, D), lambda i: (0, 0)),
        ],
        out_specs=pl.BlockSpec((_BLK, D), lambda i: (i, 0)),
        out_shape=jax.ShapeDtypeStruct((N, D), F32),
    )(aggp, z, dinv, b2r)


def _tconv_body(hp_ref, hc_ref, hn_ref, wt_ref, btr_ref, wfc_ref, bfc_ref,
                out_ref):
    i = pl.program_id(0)
    hc = hc_ref[...]
    wfc = wfc_ref[...]
    dn = (((0,), (0,)), ((), ()))
    u0 = lax.dot_general(wt_ref[0], wfc, dn, precision=_PREC,
                         preferred_element_type=F32)
    u1 = lax.dot_general(wt_ref[1], wfc, dn, precision=_PREC,
                         preferred_element_type=F32)
    u2 = lax.dot_general(wt_ref[2], wfc, dn, precision=_PREC,
                         preferred_element_type=F32)
    cst = jnp.dot(btr_ref[...], wfc, preferred_element_type=F32,
                  precision=_PREC) + bfc_ref[...]
    zrow = jnp.zeros((1, D), F32)
    first = jnp.where(i > 0, hp_ref[7:8, :], zrow)
    lastr = jnp.where(i < _GRID - 1, hn_ref[0:1, :], zrow)
    prev = jnp.concatenate([first, hc[:-1]], axis=0)
    nxt = jnp.concatenate([hc[1:], lastr], axis=0)
    out = jnp.dot(prev, u0, preferred_element_type=F32, precision=_PREC)
    out += jnp.dot(hc, u1, preferred_element_type=F32, precision=_PREC)
    out += jnp.dot(nxt, u2, preferred_element_type=F32, precision=_PREC)
    out_ref[...] = out + cst


def _tc_tconv(h, wt_t, btr, Wfc, bfcr):
    nhalo = N // 8
    return pl.pallas_call(
        _tconv_body,
        grid=(_GRID,),
        in_specs=[
            pl.BlockSpec((8, D), lambda i: (jnp.maximum(i * (_BLK // 8) - 1, 0), 0)),
            pl.BlockSpec((_BLK, D), lambda i: (i, 0)),
            pl.BlockSpec((8, D), lambda i: (jnp.minimum((i + 1) * (_BLK // 8), nhalo - 1), 0)),
            pl.BlockSpec((3, D, D), lambda i: (0, 0, 0)),
            pl.BlockSpec((1, D), lambda i: (0, 0)),
            pl.BlockSpec((D, D), lambda i: (0, 0)),
            pl.BlockSpec((1, D), lambda i: (0, 0)),
        ],
        out_specs=pl.BlockSpec((_BLK, D), lambda i: (i, 0)),
        out_shape=jax.ShapeDtypeStruct((N, D), F32),
    )(h, h, h, wt_t, btr, Wfc, bfcr)


def kernel(x, edge_index, W1, b1, W2, b2, Wt, bt, Wfc, bfc):
    src3 = edge_index[0].reshape(NC, NS, NCHUNK * CH)
    dst4 = edge_index[1].reshape(NC, NS, NCHUNK, CH)
    degp = _sc_deg(dst4)
    dinv, z1 = _tc_prep(degp, x, W1)
    agg1 = _sc_scatter(z1, src3, dst4)
    z2 = _tc_mid(agg1, z1, dinv, b1.reshape(1, D), W2)
    agg2 = _sc_scatter(z2, src3, dst4)
    h2 = _tc_act(agg2, z2, dinv, b2.reshape(1, D))
    return _tc_tconv(h2, jnp.transpose(Wt, (2, 0, 1)), bt.reshape(1, D),
                     Wfc, bfc.reshape(1, D))


# speedup vs baseline: 28.9987x; 1.0936x over previous
import jax
import jax.numpy as jnp
from jax import lax
from jax.experimental import pallas as pl
from jax.experimental.pallas import tpu as pltpu
from jax.experimental.pallas import tpu_sc as plsc

N = 10000
D = 128
E = 320000
NC = 2
NS = 16
CH = 80
NCHUNK = 125
NPAD = 10240
RPT = NPAD // NS
ZROWS = 128
F32 = jnp.float32


CH_D = 80
NCHUNK_D = 125


def _deg_body(dst_hbm, out_hbm, dacc, idx_v, ones_v, zb_v):
    c = lax.axis_index("c")
    s = lax.axis_index("s")

    def fill(r, carry):
        ones_v[r, :] = jnp.full((16,), 1.0, F32)
        return carry
    lax.fori_loop(0, CH_D, fill, 0)

    def fillz(r, carry):
        zb_v[r, :] = jnp.zeros((16,), F32)
        return carry
    lax.fori_loop(0, ZROWS, fillz, 0)

    for k in range(RPT // ZROWS):
        pltpu.sync_copy(zb_v, dacc.at[pl.ds(s * RPT + k * ZROWS, ZROWS)])
    pltpu.sync_copy(dst_hbm.at[c, s], idx_v)
    plsc.subcore_barrier()

    def chunk(j, carry):
        pltpu.sync_copy(ones_v, dacc.at[idx_v.at[j]], add=True)
        return carry
    lax.fori_loop(0, NCHUNK_D, chunk, 0)

    plsc.subcore_barrier()
    pltpu.sync_copy(dacc.at[pl.ds(s * RPT, RPT)],
                    out_hbm.at[c, pl.ds(s * RPT, RPT)])


def _sc_deg(dst4):
    return pl.kernel(
        _deg_body,
        out_type=jax.ShapeDtypeStruct((NC, NPAD, 16), F32),
        mesh=plsc.VectorSubcoreMesh(core_axis_name="c", subcore_axis_name="s"),
        scratch_types=[
            pltpu.VMEM_SHARED((NPAD, 16), F32),
            pltpu.VMEM((NCHUNK_D, CH_D), jnp.int32),
            pltpu.VMEM((CH_D, 16), F32),
            pltpu.VMEM((ZROWS, 16), F32),
        ],
    )(dst4)


def _scatter_body(z_hbm, src_hbm, dst_hbm, out_hbm, acc,
                  isv, idv, rows0, rows1, sem0, sem1):
    c = lax.axis_index("c")
    s = lax.axis_index("s")

    def fillz(r, carry):
        for k8 in range(D // 16):
            rows0[r, pl.ds(k8 * 16, 16)] = jnp.zeros((16,), F32)
        return carry
    lax.fori_loop(0, CH, fillz, 0)

    for k in range(RPT // CH):
        pltpu.sync_copy(rows0, acc.at[pl.ds(s * RPT + k * CH, CH)])
    pltpu.sync_copy(src_hbm.at[c, s], isv)
    pltpu.sync_copy(dst_hbm.at[c, s], idv)
    plsc.subcore_barrier()

    pltpu.async_copy(z_hbm.at[isv.at[pl.ds(0, CH)]], rows0, sem0)

    def pair(t, carry):
        j = 2 * t
        pltpu.async_copy(z_hbm.at[isv.at[pl.ds((j + 1) * CH, CH)]], rows1, sem1)
        pltpu.make_async_copy(z_hbm.at[isv.at[pl.ds(j * CH, CH)]], rows0, sem0).wait()
        pltpu.sync_copy(rows0, acc.at[idv.at[j]], add=True)
        pltpu.async_copy(z_hbm.at[isv.at[pl.ds((j + 2) * CH, CH)]], rows0, sem0)
        pltpu.make_async_copy(z_hbm.at[isv.at[pl.ds((j + 1) * CH, CH)]], rows1, sem1).wait()
        pltpu.sync_copy(rows1, acc.at[idv.at[j + 1]], add=True)
        return carry
    lax.fori_loop(0, (NCHUNK - 1) // 2, pair, 0)

    last = NCHUNK - 1
    pltpu.make_async_copy(z_hbm.at[isv.at[pl.ds(last * CH, CH)]], rows0, sem0).wait()
    pltpu.sync_copy(rows0, acc.at[idv.at[last]], add=True)

    plsc.subcore_barrier()
    pltpu.sync_copy(acc.at[pl.ds(s * RPT, RPT)],
                    out_hbm.at[c, pl.ds(s * RPT, RPT)])


def _sc_scatter(z, src3, dst4):
    return pl.kernel(
        _scatter_body,
        out_type=jax.ShapeDtypeStruct((NC, NPAD, D), F32),
        mesh=plsc.VectorSubcoreMesh(core_axis_name="c", subcore_axis_name="s"),
        scratch_types=[
            pltpu.VMEM_SHARED((NPAD, D), F32),
            pltpu.VMEM((NCHUNK * CH,), jnp.int32),
            pltpu.VMEM((NCHUNK, CH), jnp.int32),
            pltpu.VMEM((CH, D), F32),
            pltpu.VMEM((CH, D), F32),
            pltpu.SemaphoreType.DMA,
            pltpu.SemaphoreType.DMA,
        ],
    )(z, src3, dst4)


_BLK = 1000
_GRID = N // _BLK
_PREC = lax.Precision.DEFAULT


def _prep_body(degp_ref, x_ref, w_ref, dinv_ref, z_ref):
    deg = degp_ref[0] + degp_ref[1]
    dinv = lax.rsqrt(deg[:, 0:1] + 1.0)
    dinv_ref[...] = dinv
    z_ref[...] = jnp.dot(x_ref[...], w_ref[...],
                         preferred_element_type=F32, precision=_PREC) * dinv


def _tc_prep(degp, x, W1):
    return pl.pallas_call(
        _prep_body,
        grid=(_GRID,),
        in_specs=[
            pl.BlockSpec((NC, _BLK, 16), lambda i: (0, i, 0)),
            pl.BlockSpec((_BLK, D), lambda i: (i, 0)),
            pl.BlockSpec((D, D), lambda i: (0, 0)),
        ],
        out_specs=[
            pl.BlockSpec((_BLK, 1), lambda i: (i, 0)),
            pl.BlockSpec((_BLK, D), lambda i: (i, 0)),
        ],
        out_shape=[
            jax.ShapeDtypeStruct((N, 1), F32),
            jax.ShapeDtypeStruct((N, D), F32),
        ],
    )(degp, x, W1)


def _mid_body(aggp_ref, z_ref, dinv_ref, b_ref, w_ref, out_ref):
    dinv = dinv_ref[...]
    h = (aggp_ref[0] + aggp_ref[1] + z_ref[...]) * dinv + b_ref[...]
    h = jnp.maximum(h, 0.0)
    out_ref[...] = jnp.dot(h, w_ref[...],
                           preferred_element_type=F32, precision=_PREC) * dinv


def _tc_mid(aggp, z, dinv, b1r, W2):
    return pl.pallas_call(
        _mid_body,
        grid=(_GRID,),
        in_specs=[
            pl.BlockSpec((NC, _BLK, D), lambda i: (0, i, 0)),
            pl.BlockSpec((_BLK, D), lambda i: (i, 0)),
            pl.BlockSpec((_BLK, 1), lambda i: (i, 0)),
            pl.BlockSpec((1, D), lambda i: (0, 0)),
            pl.BlockSpec((D, D), lambda i: (0, 0)),
        ],
        out_specs=pl.BlockSpec((_BLK, D), lambda i: (i, 0)),
        out_shape=jax.ShapeDtypeStruct((N, D), F32),
    )(aggp, z, dinv, b1r, W2)


def _act_body(aggp_ref, z_ref, dinv_ref, b_ref, h_ref):
    dinv = dinv_ref[...]
    h = (aggp_ref[0] + aggp_ref[1] + z_ref[...]) * dinv + b_ref[...]
    h_ref[...] = jnp.maximum(h, 0.0)


def _tc_act(aggp, z, dinv, b2r):
    return pl.pallas_call(
        _act_body,
        grid=(_GRID,),
        in_specs=[
            pl.BlockSpec((NC, _BLK, D), lambda i: (0, i, 0)),
            pl.BlockSpec((_BLK, D), lambda i: (i, 0)),
            pl.BlockSpec((_BLK, 1), lambda i: (i, 0)),
            pl.BlockSpec((1, D), lambda i: (0, 0)),
        ],
        out_specs=pl.BlockSpec((_BLK, D), lambda i: (i, 0)),
        out_shape=jax.ShapeDtypeStruct((N, D), F32),
    )(aggp, z, dinv, b2r)


def _tconv_body(hp_ref, hc_ref, hn_ref, wt_ref, btr_ref, wfc_ref, bfc_ref,
                out_ref):
    i = pl.program_id(0)
    hc = hc_ref[...]
    wfc = wfc_ref[...]
    dn = (((0,), (0,)), ((), ()))
    u0 = lax.dot_general(wt_ref[0], wfc, dn, precision=_PREC,
                         preferred_element_type=F32)
    u1 = lax.dot_general(wt_ref[1], wfc, dn, precision=_PREC,
                         preferred_element_type=F32)
    u2 = lax.dot_general(wt_ref[2], wfc, dn, precision=_PREC,
                         preferred_element_type=F32)
    cst = jnp.dot(btr_ref[...], wfc, preferred_element_type=F32,
                  precision=_PREC) + bfc_ref[...]
    zrow = jnp.zeros((1, D), F32)
    first = jnp.where(i > 0, hp_ref[7:8, :], zrow)
    lastr = jnp.where(i < _GRID - 1, hn_ref[0:1, :], zrow)
    prev = jnp.concatenate([first, hc[:-1]], axis=0)
    nxt = jnp.concatenate([hc[1:], lastr], axis=0)
    out = jnp.dot(prev, u0, preferred_element_type=F32, precision=_PREC)
    out += jnp.dot(hc, u1, preferred_element_type=F32, precision=_PREC)
    out += jnp.dot(nxt, u2, preferred_element_type=F32, precision=_PREC)
    out_ref[...] = out + cst


def _tc_tconv(h, wt_t, btr, Wfc, bfcr):
    nhalo = N // 8
    return pl.pallas_call(
        _tconv_body,
        grid=(_GRID,),
        in_specs=[
            pl.BlockSpec((8, D), lambda i: (jnp.maximum(i * (_BLK // 8) - 1, 0), 0)),
            pl.BlockSpec((_BLK, D), lambda i: (i, 0)),
            pl.BlockSpec((8, D), lambda i: (jnp.minimum((i + 1) * (_BLK // 8), nhalo - 1), 0)),
            pl.BlockSpec((3, D, D), lambda i: (0, 0, 0)),
            pl.BlockSpec((1, D), lambda i: (0, 0)),
            pl.BlockSpec((D, D), lambda i: (0, 0)),
            pl.BlockSpec((1, D), lambda i: (0, 0)),
        ],
        out_specs=pl.BlockSpec((_BLK, D), lambda i: (i, 0)),
        out_shape=jax.ShapeDtypeStruct((N, D), F32),
    )(h, h, h, wt_t, btr, Wfc, bfcr)


def kernel(x, edge_index, W1, b1, W2, b2, Wt, bt, Wfc, bfc):
    src3 = edge_index[0].reshape(NC, NS, NCHUNK * CH)
    dst4 = edge_index[1].reshape(NC, NS, NCHUNK, CH)
    degp = _sc_deg(dst4)
    dinv, z1 = _tc_prep(degp, x, W1)
    agg1 = _sc_scatter(z1, src3, dst4)
    z2 = _tc_mid(agg1, z1, dinv, b1.reshape(1, D), W2)
    agg2 = _sc_scatter(z2, src3, dst4)
    h2 = _tc_act(agg2, z2, dinv, b2.reshape(1, D))
    return _tc_tconv(h2, jnp.transpose(Wt, (2, 0, 1)), bt.reshape(1, D),
                     Wfc, bfc.reshape(1, D))


# speedup vs baseline: 29.7790x; 1.0269x over previous
import jax
import jax.numpy as jnp
from jax import lax
from jax.experimental import pallas as pl
from jax.experimental.pallas import tpu as pltpu
from jax.experimental.pallas import tpu_sc as plsc

N = 10000
D = 128
E = 320000
NC = 2
NS = 16
CH = 80
NCHUNK = 125
NPAD = 10240
RPT = NPAD // NS
ZROWS = 128
F32 = jnp.float32


CH_D = 80
NCHUNK_D = 125


def _deg_body(dst_hbm, out_hbm, dacc, idx_v, ones_v, zb_v):
    c = lax.axis_index("c")
    s = lax.axis_index("s")

    def fill(r, carry):
        ones_v[r, :] = jnp.full((16,), 1.0, F32)
        return carry
    lax.fori_loop(0, CH_D, fill, 0)

    def fillz(r, carry):
        zb_v[r, :] = jnp.zeros((16,), F32)
        return carry
    lax.fori_loop(0, ZROWS, fillz, 0)

    for k in range(RPT // ZROWS):
        pltpu.sync_copy(zb_v, dacc.at[pl.ds(s * RPT + k * ZROWS, ZROWS)])
    pltpu.sync_copy(dst_hbm.at[c, s], idx_v)
    plsc.subcore_barrier()

    def chunk(j, carry):
        pltpu.sync_copy(ones_v, dacc.at[idx_v.at[j]], add=True)
        return carry
    lax.fori_loop(0, NCHUNK_D, chunk, 0)

    plsc.subcore_barrier()
    pltpu.sync_copy(dacc.at[pl.ds(s * RPT, RPT)],
                    out_hbm.at[c, pl.ds(s * RPT, RPT)])


def _sc_deg(dst4):
    return pl.kernel(
        _deg_body,
        out_type=jax.ShapeDtypeStruct((NC, NPAD, 16), F32),
        mesh=plsc.VectorSubcoreMesh(core_axis_name="c", subcore_axis_name="s"),
        scratch_types=[
            pltpu.VMEM_SHARED((NPAD, 16), F32),
            pltpu.VMEM((NCHUNK_D, CH_D), jnp.int32),
            pltpu.VMEM((CH_D, 16), F32),
            pltpu.VMEM((ZROWS, 16), F32),
        ],
    )(dst4)


def _scatter_body(z_hbm, src_hbm, dst_hbm, out_hbm, acc,
                  isv, idv, rows0, rows1, sem0, sem1):
    c = lax.axis_index("c")
    s = lax.axis_index("s")

    def fillz(r, carry):
        for k8 in range(D // 16):
            rows0[r, pl.ds(k8 * 16, 16)] = jnp.zeros((16,), F32)
        return carry
    lax.fori_loop(0, CH, fillz, 0)

    for k in range(RPT // CH):
        pltpu.sync_copy(rows0, acc.at[pl.ds(s * RPT + k * CH, CH)])
    pltpu.sync_copy(src_hbm.at[c, s], isv)
    pltpu.sync_copy(dst_hbm.at[c, s], idv)
    plsc.subcore_barrier()

    pltpu.async_copy(z_hbm.at[isv.at[pl.ds(0, CH)]], rows0, sem0)

    def pair(t, carry):
        j = 2 * t
        pltpu.async_copy(z_hbm.at[isv.at[pl.ds((j + 1) * CH, CH)]], rows1, sem1)
        pltpu.make_async_copy(z_hbm.at[isv.at[pl.ds(j * CH, CH)]], rows0, sem0).wait()
        pltpu.sync_copy(rows0, acc.at[idv.at[j]], add=True)
        pltpu.async_copy(z_hbm.at[isv.at[pl.ds((j + 2) * CH, CH)]], rows0, sem0)
        pltpu.make_async_copy(z_hbm.at[isv.at[pl.ds((j + 1) * CH, CH)]], rows1, sem1).wait()
        pltpu.sync_copy(rows1, acc.at[idv.at[j + 1]], add=True)
        return carry
    lax.fori_loop(0, (NCHUNK - 1) // 2, pair, 0)

    last = NCHUNK - 1
    pltpu.make_async_copy(z_hbm.at[isv.at[pl.ds(last * CH, CH)]], rows0, sem0).wait()
    pltpu.sync_copy(rows0, acc.at[idv.at[last]], add=True)

    plsc.subcore_barrier()
    pltpu.sync_copy(acc.at[pl.ds(s * RPT, RPT)],
                    out_hbm.at[c, pl.ds(s * RPT, RPT)])


def _sc_scatter(z, src3, dst4):
    return pl.kernel(
        _scatter_body,
        out_type=jax.ShapeDtypeStruct((NC, NPAD, D), F32),
        mesh=plsc.VectorSubcoreMesh(core_axis_name="c", subcore_axis_name="s"),
        scratch_types=[
            pltpu.VMEM_SHARED((NPAD, D), F32),
            pltpu.VMEM((NCHUNK * CH,), jnp.int32),
            pltpu.VMEM((NCHUNK, CH), jnp.int32),
            pltpu.VMEM((CH, D), F32),
            pltpu.VMEM((CH, D), F32),
            pltpu.SemaphoreType.DMA,
            pltpu.SemaphoreType.DMA,
        ],
    )(z, src3, dst4)


_BLK = 1000
_GRID = N // _BLK
_PREC = lax.Precision.DEFAULT


def _prep_body(degp_ref, x_ref, w_ref, dinv_ref, z_ref):
    deg = degp_ref[0] + degp_ref[1]
    dinv = lax.rsqrt(deg[:, 0:1] + 1.0)
    dinv_ref[...] = jnp.broadcast_to(dinv, (dinv.shape[0], 8))
    z_ref[...] = jnp.dot(x_ref[...], w_ref[...],
                         preferred_element_type=F32, precision=_PREC) * dinv


def _tc_prep(degp, x, W1):
    return pl.pallas_call(
        _prep_body,
        grid=(_GRID,),
        in_specs=[
            pl.BlockSpec((NC, _BLK, 16), lambda i: (0, i, 0)),
            pl.BlockSpec((_BLK, D), lambda i: (i, 0)),
            pl.BlockSpec((D, D), lambda i: (0, 0)),
        ],
        out_specs=[
            pl.BlockSpec((_BLK, 8), lambda i: (i, 0)),
            pl.BlockSpec((_BLK, D), lambda i: (i, 0)),
        ],
        out_shape=[
            jax.ShapeDtypeStruct((N, 8), F32),
            jax.ShapeDtypeStruct((N, D), F32),
        ],
    )(degp, x, W1)


def _mid_body(aggp_ref, z_ref, dinv_ref, b_ref, w_ref, out_ref):
    dinv = dinv_ref[:, 0:1]
    h = (aggp_ref[0] + aggp_ref[1] + z_ref[...]) * dinv + b_ref[...]
    h = jnp.maximum(h, 0.0)
    out_ref[...] = jnp.dot(h, w_ref[...],
                           preferred_element_type=F32, precision=_PREC) * dinv


def _tc_mid(aggp, z, dinv, b1r, W2):
    return pl.pallas_call(
        _mid_body,
        grid=(_GRID,),
        in_specs=[
            pl.BlockSpec((NC, _BLK, D), lambda i: (0, i, 0)),
            pl.BlockSpec((_BLK, D), lambda i: (i, 0)),
            pl.BlockSpec((_BLK, 8), lambda i: (i, 0)),
            pl.BlockSpec((1, D), lambda i: (0, 0)),
            pl.BlockSpec((D, D), lambda i: (0, 0)),
        ],
        out_specs=pl.BlockSpec((_BLK, D), lambda i: (i, 0)),
        out_shape=jax.ShapeDtypeStruct((N, D), F32),
    )(aggp, z, dinv, b1r, W2)


def _fin_body(a0p_ref, a1p_ref, zp_ref, dp_ref,
              ac_ref, zc_ref, dc_ref,
              a0n_ref, a1n_ref, zn_ref, dn8_ref,
              b_ref, wt_ref, btr_ref, wfc_ref, bfc_ref, out_ref):
    i = pl.program_id(0)
    b = b_ref[...]
    hc = jnp.maximum(
        (ac_ref[0] + ac_ref[1] + zc_ref[...]) * dc_ref[:, 0:1] + b, 0.0)
    hp = jnp.maximum(
        (a0p_ref[7:8] + a1p_ref[7:8] + zp_ref[7:8]) * dp_ref[7:8, 0:1] + b,
        0.0)
    hn = jnp.maximum(
        (a0n_ref[0:1] + a1n_ref[0:1] + zn_ref[0:1]) * dn8_ref[0:1, 0:1] + b,
        0.0)
    wfc = wfc_ref[...]
    dn = (((0,), (0,)), ((), ()))
    u0 = lax.dot_general(wt_ref[0], wfc, dn, precision=_PREC,
                         preferred_element_type=F32)
    u1 = lax.dot_general(wt_ref[1], wfc, dn, precision=_PREC,
                         preferred_element_type=F32)
    u2 = lax.dot_general(wt_ref[2], wfc, dn, precision=_PREC,
                         preferred_element_type=F32)
    cst = jnp.dot(btr_ref[...], wfc, preferred_element_type=F32,
                  precision=_PREC) + bfc_ref[...]
    zrow = jnp.zeros((1, D), F32)
    first = jnp.where(i > 0, hp, zrow)
    lastr = jnp.where(i < _GRID - 1, hn, zrow)
    prev = jnp.concatenate([first, hc[:-1]], axis=0)
    nxt = jnp.concatenate([hc[1:], lastr], axis=0)
    out = jnp.dot(prev, u0, preferred_element_type=F32, precision=_PREC)
    out += jnp.dot(hc, u1, preferred_element_type=F32, precision=_PREC)
    out += jnp.dot(nxt, u2, preferred_element_type=F32, precision=_PREC)
    out_ref[...] = out + cst


def _tc_final(aggp, z, dinv8, b2r, wt_t, btr, Wfc, bfcr):
    aggf = aggp.reshape(NC * NPAD, D)
    h8 = _BLK // 8
    pmap = lambda i: (jnp.maximum(i * h8 - 1, 0), 0)
    nmap = lambda i: (jnp.minimum((i + 1) * h8, N // 8 - 1), 0)
    pmap1 = lambda i: (NPAD // 8 + jnp.maximum(i * h8 - 1, 0), 0)
    nmap1 = lambda i: (NPAD // 8 + jnp.minimum((i + 1) * h8, N // 8 - 1), 0)
    return pl.pallas_call(
        _fin_body,
        grid=(_GRID,),
        in_specs=[
            pl.BlockSpec((8, D), pmap),
            pl.BlockSpec((8, D), pmap1),
            pl.BlockSpec((8, D), pmap),
            pl.BlockSpec((8, 8), pmap),
            pl.BlockSpec((NC, _BLK, D), lambda i: (0, i, 0)),
            pl.BlockSpec((_BLK, D), lambda i: (i, 0)),
            pl.BlockSpec((_BLK, 8), lambda i: (i, 0)),
            pl.BlockSpec((8, D), nmap),
            pl.BlockSpec((8, D), nmap1),
            pl.BlockSpec((8, D), nmap),
            pl.BlockSpec((8, 8), nmap),
            pl.BlockSpec((1, D), lambda i: (0, 0)),
            pl.BlockSpec((3, D, D), lambda i: (0, 0, 0)),
            pl.BlockSpec((1, D), lambda i: (0, 0)),
            pl.BlockSpec((D, D), lambda i: (0, 0)),
            pl.BlockSpec((1, D), lambda i: (0, 0)),
        ],
        out_specs=pl.BlockSpec((_BLK, D), lambda i: (i, 0)),
        out_shape=jax.ShapeDtypeStruct((N, D), F32),
    )(aggf, aggf, z, dinv8,
      aggp, z, dinv8,
      aggf, aggf, z, dinv8,
      b2r, wt_t, btr, Wfc, bfcr)


def kernel(x, edge_index, W1, b1, W2, b2, Wt, bt, Wfc, bfc):
    src3 = edge_index[0].reshape(NC, NS, NCHUNK * CH)
    dst4 = edge_index[1].reshape(NC, NS, NCHUNK, CH)
    degp = _sc_deg(dst4)
    dinv, z1 = _tc_prep(degp, x, W1)
    agg1 = _sc_scatter(z1, src3, dst4)
    z2 = _tc_mid(agg1, z1, dinv, b1.reshape(1, D), W2)
    agg2 = _sc_scatter(z2, src3, dst4)
    return _tc_final(agg2, z2, dinv, b2.reshape(1, D),
                     jnp.transpose(Wt, (2, 0, 1)), bt.reshape(1, D),
                     Wfc, bfc.reshape(1, D))


# speedup vs baseline: 29.8045x; 1.0009x over previous
import jax
import jax.numpy as jnp
from jax import lax
from jax.experimental import pallas as pl
from jax.experimental.pallas import tpu as pltpu
from jax.experimental.pallas import tpu_sc as plsc

N = 10000
D = 128
E = 320000
NC = 2
NS = 16
CH = 80
NCHUNK = 125
NPAD = 10240
RPT = NPAD // NS
ZROWS = 128
F32 = jnp.float32


CH_D = 128
NCHUNK_D = 80


def _deg_body(dst_hbm, out_hbm, dacc, idx_v, ones_v, zb_v):
    c = lax.axis_index("c")
    s = lax.axis_index("s")

    def fill(r, carry):
        ones_v[r, :] = jnp.full((16,), 1.0, F32)
        return carry
    lax.fori_loop(0, CH_D, fill, 0)

    def fillz(r, carry):
        zb_v[r, :] = jnp.zeros((16,), F32)
        return carry
    lax.fori_loop(0, ZROWS, fillz, 0)

    for k in range(RPT // ZROWS):
        pltpu.sync_copy(zb_v, dacc.at[pl.ds(s * RPT + k * ZROWS, ZROWS)])
    pltpu.sync_copy(dst_hbm.at[c, s], idx_v)
    plsc.subcore_barrier()

    def chunk(j, carry):
        pltpu.sync_copy(ones_v, dacc.at[idx_v.at[j]], add=True)
        return carry
    lax.fori_loop(0, NCHUNK_D, chunk, 0)

    plsc.subcore_barrier()
    pltpu.sync_copy(dacc.at[pl.ds(s * RPT, RPT)],
                    out_hbm.at[c, pl.ds(s * RPT, RPT)])


def _sc_deg(dst4):
    return pl.kernel(
        _deg_body,
        out_type=jax.ShapeDtypeStruct((NC, NPAD, 16), F32),
        mesh=plsc.VectorSubcoreMesh(core_axis_name="c", subcore_axis_name="s"),
        scratch_types=[
            pltpu.VMEM_SHARED((NPAD, 16), F32),
            pltpu.VMEM((NCHUNK_D, CH_D), jnp.int32),
            pltpu.VMEM((CH_D, 16), F32),
            pltpu.VMEM((ZROWS, 16), F32),
        ],
    )(dst4)


def _scatter_body(z_hbm, src_hbm, dst_hbm, out_hbm, acc,
                  isv, idv, rows0, rows1, sem0, sem1):
    c = lax.axis_index("c")
    s = lax.axis_index("s")

    def fillz(r, carry):
        for k8 in range(D // 16):
            rows0[r, pl.ds(k8 * 16, 16)] = jnp.zeros((16,), F32)
        return carry
    lax.fori_loop(0, CH, fillz, 0)

    for k in range(RPT // CH):
        pltpu.sync_copy(rows0, acc.at[pl.ds(s * RPT + k * CH, CH)])
    pltpu.sync_copy(src_hbm.at[c, s], isv)
    pltpu.sync_copy(dst_hbm.at[c, s], idv)
    plsc.subcore_barrier()

    pltpu.async_copy(z_hbm.at[isv.at[pl.ds(0, CH)]], rows0, sem0)

    def pair(t, carry):
        j = 2 * t
        pltpu.async_copy(z_hbm.at[isv.at[pl.ds((j + 1) * CH, CH)]], rows1, sem1)
        pltpu.make_async_copy(z_hbm.at[isv.at[pl.ds(j * CH, CH)]], rows0, sem0).wait()
        pltpu.sync_copy(rows0, acc.at[idv.at[j]], add=True)
        pltpu.async_copy(z_hbm.at[isv.at[pl.ds((j + 2) * CH, CH)]], rows0, sem0)
        pltpu.make_async_copy(z_hbm.at[isv.at[pl.ds((j + 1) * CH, CH)]], rows1, sem1).wait()
        pltpu.sync_copy(rows1, acc.at[idv.at[j + 1]], add=True)
        return carry
    lax.fori_loop(0, (NCHUNK - 1) // 2, pair, 0)

    last = NCHUNK - 1
    pltpu.make_async_copy(z_hbm.at[isv.at[pl.ds(last * CH, CH)]], rows0, sem0).wait()
    pltpu.sync_copy(rows0, acc.at[idv.at[last]], add=True)

    plsc.subcore_barrier()
    pltpu.sync_copy(acc.at[pl.ds(s * RPT, RPT)],
                    out_hbm.at[c, pl.ds(s * RPT, RPT)])


def _sc_scatter(z, src3, dst4):
    return pl.kernel(
        _scatter_body,
        out_type=jax.ShapeDtypeStruct((NC, NPAD, D), F32),
        mesh=plsc.VectorSubcoreMesh(core_axis_name="c", subcore_axis_name="s"),
        scratch_types=[
            pltpu.VMEM_SHARED((NPAD, D), F32),
            pltpu.VMEM((NCHUNK * CH,), jnp.int32),
            pltpu.VMEM((NCHUNK, CH), jnp.int32),
            pltpu.VMEM((CH, D), F32),
            pltpu.VMEM((CH, D), F32),
            pltpu.SemaphoreType.DMA,
            pltpu.SemaphoreType.DMA,
        ],
    )(z, src3, dst4)


_BLK = 1000
_GRID = N // _BLK
_PREC = lax.Precision.DEFAULT


def _prep_body(degp_ref, x_ref, w_ref, dinv_ref, z_ref):
    deg = degp_ref[0] + degp_ref[1]
    dinv = lax.rsqrt(deg[:, 0:1] + 1.0)
    dinv_ref[...] = jnp.broadcast_to(dinv, (dinv.shape[0], 8))
    z_ref[...] = jnp.dot(x_ref[...], w_ref[...],
                         preferred_element_type=F32, precision=_PREC) * dinv


def _tc_prep(degp, x, W1):
    return pl.pallas_call(
        _prep_body,
        grid=(_GRID,),
        in_specs=[
            pl.BlockSpec((NC, _BLK, 16), lambda i: (0, i, 0)),
            pl.BlockSpec((_BLK, D), lambda i: (i, 0)),
            pl.BlockSpec((D, D), lambda i: (0, 0)),
        ],
        out_specs=[
            pl.BlockSpec((_BLK, 8), lambda i: (i, 0)),
            pl.BlockSpec((_BLK, D), lambda i: (i, 0)),
        ],
        out_shape=[
            jax.ShapeDtypeStruct((N, 8), F32),
            jax.ShapeDtypeStruct((N, D), F32),
        ],
    )(degp, x, W1)


def _mid_body(aggp_ref, z_ref, dinv_ref, b_ref, w_ref, out_ref):
    dinv = dinv_ref[:, 0:1]
    h = (aggp_ref[0] + aggp_ref[1] + z_ref[...]) * dinv + b_ref[...]
    h = jnp.maximum(h, 0.0)
    out_ref[...] = jnp.dot(h, w_ref[...],
                           preferred_element_type=F32, precision=_PREC) * dinv


def _tc_mid(aggp, z, dinv, b1r, W2):
    return pl.pallas_call(
        _mid_body,
        grid=(_GRID,),
        in_specs=[
            pl.BlockSpec((NC, _BLK, D), lambda i: (0, i, 0)),
            pl.BlockSpec((_BLK, D), lambda i: (i, 0)),
            pl.BlockSpec((_BLK, 8), lambda i: (i, 0)),
            pl.BlockSpec((1, D), lambda i: (0, 0)),
            pl.BlockSpec((D, D), lambda i: (0, 0)),
        ],
        out_specs=pl.BlockSpec((_BLK, D), lambda i: (i, 0)),
        out_shape=jax.ShapeDtypeStruct((N, D), F32),
    )(aggp, z, dinv, b1r, W2)


def _fin_body(a0p_ref, a1p_ref, zp_ref, dp_ref,
              ac_ref, zc_ref, dc_ref,
              a0n_ref, a1n_ref, zn_ref, dn8_ref,
              b_ref, wt_ref, btr_ref, wfc_ref, bfc_ref, out_ref):
    i = pl.program_id(0)
    b = b_ref[...]
    hc = jnp.maximum(
        (ac_ref[0] + ac_ref[1] + zc_ref[...]) * dc_ref[:, 0:1] + b, 0.0)
    hp = jnp.maximum(
        (a0p_ref[7:8] + a1p_ref[7:8] + zp_ref[7:8]) * dp_ref[7:8, 0:1] + b,
        0.0)
    hn = jnp.maximum(
        (a0n_ref[0:1] + a1n_ref[0:1] + zn_ref[0:1]) * dn8_ref[0:1, 0:1] + b,
        0.0)
    wfc = wfc_ref[...]
    dn = (((0,), (0,)), ((), ()))
    u0 = lax.dot_general(wt_ref[0], wfc, dn, precision=_PREC,
                         preferred_element_type=F32)
    u1 = lax.dot_general(wt_ref[1], wfc, dn, precision=_PREC,
                         preferred_element_type=F32)
    u2 = lax.dot_general(wt_ref[2], wfc, dn, precision=_PREC,
                         preferred_element_type=F32)
    cst = jnp.dot(btr_ref[...], wfc, preferred_element_type=F32,
                  precision=_PREC) + bfc_ref[...]
    zrow = jnp.zeros((1, D), F32)
    first = jnp.where(i > 0, hp, zrow)
    lastr = jnp.where(i < _GRID - 1, hn, zrow)
    prev = jnp.concatenate([first, hc[:-1]], axis=0)
    nxt = jnp.concatenate([hc[1:], lastr], axis=0)
    out = jnp.dot(prev, u0, preferred_element_type=F32, precision=_PREC)
    out += jnp.dot(hc, u1, preferred_element_type=F32, precision=_PREC)
    out += jnp.dot(nxt, u2, preferred_element_type=F32, precision=_PREC)
    out_ref[...] = out + cst


def _tc_final(aggp, z, dinv8, b2r, wt_t, btr, Wfc, bfcr):
    aggf = aggp.reshape(NC * NPAD, D)
    h8 = _BLK // 8
    pmap = lambda i: (jnp.maximum(i * h8 - 1, 0), 0)
    nmap = lambda i: (jnp.minimum((i + 1) * h8, N // 8 - 1), 0)
    pmap1 = lambda i: (NPAD // 8 + jnp.maximum(i * h8 - 1, 0), 0)
    nmap1 = lambda i: (NPAD // 8 + jnp.minimum((i + 1) * h8, N // 8 - 1), 0)
    return pl.pallas_call(
        _fin_body,
        grid=(_GRID,),
        in_specs=[
            pl.BlockSpec((8, D), pmap),
            pl.BlockSpec((8, D), pmap1),
            pl.BlockSpec((8, D), pmap),
            pl.BlockSpec((8, 8), pmap),
            pl.BlockSpec((NC, _BLK, D), lambda i: (0, i, 0)),
            pl.BlockSpec((_BLK, D), lambda i: (i, 0)),
            pl.BlockSpec((_BLK, 8), lambda i: (i, 0)),
            pl.BlockSpec((8, D), nmap),
            pl.BlockSpec((8, D), nmap1),
            pl.BlockSpec((8, D), nmap),
            pl.BlockSpec((8, 8), nmap),
            pl.BlockSpec((1, D), lambda i: (0, 0)),
            pl.BlockSpec((3, D, D), lambda i: (0, 0, 0)),
            pl.BlockSpec((1, D), lambda i: (0, 0)),
            pl.BlockSpec((D, D), lambda i: (0, 0)),
            pl.BlockSpec((1, D), lambda i: (0, 0)),
        ],
        out_specs=pl.BlockSpec((_BLK, D), lambda i: (i, 0)),
        out_shape=jax.ShapeDtypeStruct((N, D), F32),
    )(aggf, aggf, z, dinv8,
      aggp, z, dinv8,
      aggf, aggf, z, dinv8,
      b2r, wt_t, btr, Wfc, bfcr)


def kernel(x, edge_index, W1, b1, W2, b2, Wt, bt, Wfc, bfc):
    src3 = edge_index[0].reshape(NC, NS, NCHUNK * CH)
    dst4 = edge_index[1].reshape(NC, NS, NCHUNK, CH)
    dst_pad = jnp.concatenate(
        [edge_index[1].reshape(NC * NS, E // (NC * NS)),
         jnp.full((NC * NS, NCHUNK_D * CH_D - E // (NC * NS)), NPAD - 1,
                  jnp.int32)], axis=1).reshape(NC, NS, NCHUNK_D, CH_D)
    degp = _sc_deg(dst_pad)
    dinv, z1 = _tc_prep(degp, x, W1)
    agg1 = _sc_scatter(z1, src3, dst4)
    z2 = _tc_mid(agg1, z1, dinv, b1.reshape(1, D), W2)
    agg2 = _sc_scatter(z2, src3, dst4)
    return _tc_final(agg2, z2, dinv, b2.reshape(1, D),
                     jnp.transpose(Wt, (2, 0, 1)), bt.reshape(1, D),
                     Wfc, bfc.reshape(1, D))


# speedup vs baseline: 30.3825x; 1.0194x over previous
import jax
import jax.numpy as jnp
from jax import lax
from jax.experimental import pallas as pl
from jax.experimental.pallas import tpu as pltpu
from jax.experimental.pallas import tpu_sc as plsc

N = 10000
D = 128
E = 320000
NC = 2
NS = 16
CH = 80
NCHUNK = 125
NPAD = 10240
RPT = NPAD // NS
ZROWS = 128
F32 = jnp.float32


CH_D = 128
NCHUNK_D = 80


def _deg_body(dst_hbm, out_hbm, dacc, idx_v, ones_v, zb_v):
    c = lax.axis_index("c")
    s = lax.axis_index("s")

    def fill(r, carry):
        ones_v[r, :] = jnp.full((16,), 1.0, F32)
        return carry
    lax.fori_loop(0, CH_D, fill, 0)

    def fillz(r, carry):
        zb_v[r, :] = jnp.zeros((16,), F32)
        return carry
    lax.fori_loop(0, ZROWS, fillz, 0)

    for k in range(RPT // ZROWS):
        pltpu.sync_copy(zb_v, dacc.at[pl.ds(s * RPT + k * ZROWS, ZROWS)])
    pltpu.sync_copy(dst_hbm.at[c, s], idx_v)
    plsc.subcore_barrier()

    def chunk(j, carry):
        pltpu.sync_copy(ones_v, dacc.at[idx_v.at[j]], add=True)
        return carry
    lax.fori_loop(0, NCHUNK_D, chunk, 0)

    plsc.subcore_barrier()
    pltpu.sync_copy(dacc.at[pl.ds(s * RPT, RPT)],
                    out_hbm.at[c, pl.ds(s * RPT, RPT)])


def _sc_deg(dst4):
    return pl.kernel(
        _deg_body,
        out_type=jax.ShapeDtypeStruct((NC, NPAD, 16), F32),
        mesh=plsc.VectorSubcoreMesh(core_axis_name="c", subcore_axis_name="s"),
        scratch_types=[
            pltpu.VMEM_SHARED((NPAD, 16), F32),
            pltpu.VMEM((NCHUNK_D, CH_D), jnp.int32),
            pltpu.VMEM((CH_D, 16), F32),
            pltpu.VMEM((ZROWS, 16), F32),
        ],
    )(dst4)


def _scatter_body(z_hbm, src_hbm, dst_hbm, out_hbm, acc,
                  isv, idv, rows0, rows1, sem0, sem1):
    c = lax.axis_index("c")
    s = lax.axis_index("s")

    pltpu.sync_copy(src_hbm.at[c, s], isv)
    pltpu.sync_copy(dst_hbm.at[c, s], idv)
    pltpu.async_copy(z_hbm.at[isv.at[pl.ds(0, CH)]], rows0, sem0)

    def fillz(r, carry):
        for k8 in range(D // 16):
            rows1[r, pl.ds(k8 * 16, 16)] = jnp.zeros((16,), F32)
        return carry
    lax.fori_loop(0, CH, fillz, 0)

    for k in range(RPT // CH):
        pltpu.sync_copy(rows1, acc.at[pl.ds(s * RPT + k * CH, CH)])
    plsc.subcore_barrier()
    pltpu.async_copy(z_hbm.at[isv.at[pl.ds(CH, CH)]], rows1, sem1)

    def pair(t, carry):
        j = 2 * t
        pltpu.make_async_copy(z_hbm.at[isv.at[pl.ds(j * CH, CH)]], rows0, sem0).wait()
        pltpu.sync_copy(rows0, acc.at[idv.at[j]], add=True)
        pltpu.async_copy(z_hbm.at[isv.at[pl.ds((j + 2) * CH, CH)]], rows0, sem0)
        pltpu.make_async_copy(z_hbm.at[isv.at[pl.ds((j + 1) * CH, CH)]], rows1, sem1).wait()
        pltpu.sync_copy(rows1, acc.at[idv.at[j + 1]], add=True)

        @pl.when(t < (NCHUNK - 1) // 2 - 1)
        def _():
            pltpu.async_copy(z_hbm.at[isv.at[pl.ds((j + 3) * CH, CH)]], rows1,
                             sem1)
        return carry
    lax.fori_loop(0, (NCHUNK - 1) // 2, pair, 0)

    last = NCHUNK - 1
    pltpu.make_async_copy(z_hbm.at[isv.at[pl.ds(last * CH, CH)]], rows0, sem0).wait()
    pltpu.sync_copy(rows0, acc.at[idv.at[last]], add=True)

    plsc.subcore_barrier()
    pltpu.sync_copy(acc.at[pl.ds(s * RPT, RPT)],
                    out_hbm.at[c, pl.ds(s * RPT, RPT)])


def _sc_scatter(z, src3, dst4):
    return pl.kernel(
        _scatter_body,
        out_type=jax.ShapeDtypeStruct((NC, NPAD, D), F32),
        mesh=plsc.VectorSubcoreMesh(core_axis_name="c", subcore_axis_name="s"),
        scratch_types=[
            pltpu.VMEM_SHARED((NPAD, D), F32),
            pltpu.VMEM((NCHUNK * CH,), jnp.int32),
            pltpu.VMEM((NCHUNK, CH), jnp.int32),
            pltpu.VMEM((CH, D), F32),
            pltpu.VMEM((CH, D), F32),
            pltpu.SemaphoreType.DMA,
            pltpu.SemaphoreType.DMA,
        ],
    )(z, src3, dst4)


_BLK = 2000
_GRID = N // _BLK
_FBLK = 1000
_FGRID = N // _FBLK
_PREC = lax.Precision.DEFAULT


def _xw_body(x_ref, w_ref, xw_ref):
    xw_ref[...] = jnp.dot(x_ref[...], w_ref[...],
                          preferred_element_type=F32, precision=_PREC)


def _tc_xw(x, W1):
    return pl.pallas_call(
        _xw_body,
        grid=(_GRID,),
        in_specs=[
            pl.BlockSpec((_BLK, D), lambda i: (i, 0)),
            pl.BlockSpec((D, D), lambda i: (0, 0)),
        ],
        out_specs=pl.BlockSpec((_BLK, D), lambda i: (i, 0)),
        out_shape=jax.ShapeDtypeStruct((N, D), F32),
    )(x, W1)


def _prep_body(degp_ref, xw_ref, dinv_ref, z_ref):
    deg = degp_ref[0] + degp_ref[1]
    dinv = lax.rsqrt(deg[:, 0:1] + 1.0)
    dinv_ref[...] = jnp.broadcast_to(dinv, (dinv.shape[0], 8))
    z_ref[...] = xw_ref[...] * dinv


def _tc_prep(degp, xw):
    return pl.pallas_call(
        _prep_body,
        grid=(_GRID,),
        in_specs=[
            pl.BlockSpec((NC, _BLK, 16), lambda i: (0, i, 0)),
            pl.BlockSpec((_BLK, D), lambda i: (i, 0)),
        ],
        out_specs=[
            pl.BlockSpec((_BLK, 8), lambda i: (i, 0)),
            pl.BlockSpec((_BLK, D), lambda i: (i, 0)),
        ],
        out_shape=[
            jax.ShapeDtypeStruct((N, 8), F32),
            jax.ShapeDtypeStruct((N, D), F32),
        ],
    )(degp, xw)


def _mid_body(aggp_ref, z_ref, dinv_ref, b_ref, w_ref, out_ref):
    dinv = dinv_ref[:, 0:1]
    h = (aggp_ref[0] + aggp_ref[1] + z_ref[...]) * dinv + b_ref[...]
    h = jnp.maximum(h, 0.0)
    out_ref[...] = jnp.dot(h, w_ref[...],
                           preferred_element_type=F32, precision=_PREC) * dinv


def _tc_mid(aggp, z, dinv, b1r, W2):
    return pl.pallas_call(
        _mid_body,
        grid=(_GRID,),
        in_specs=[
            pl.BlockSpec((NC, _BLK, D), lambda i: (0, i, 0)),
            pl.BlockSpec((_BLK, D), lambda i: (i, 0)),
            pl.BlockSpec((_BLK, 8), lambda i: (i, 0)),
            pl.BlockSpec((1, D), lambda i: (0, 0)),
            pl.BlockSpec((D, D), lambda i: (0, 0)),
        ],
        out_specs=pl.BlockSpec((_BLK, D), lambda i: (i, 0)),
        out_shape=jax.ShapeDtypeStruct((N, D), F32),
    )(aggp, z, dinv, b1r, W2)


def _fin_body(a0p_ref, a1p_ref, zp_ref, dp_ref,
              ac_ref, zc_ref, dc_ref,
              a0n_ref, a1n_ref, zn_ref, dn8_ref,
              b_ref, wt_ref, btr_ref, wfc_ref, bfc_ref, out_ref):
    i = pl.program_id(0)
    b = b_ref[...]
    hc = jnp.maximum(
        (ac_ref[0] + ac_ref[1] + zc_ref[...]) * dc_ref[:, 0:1] + b, 0.0)
    hp = jnp.maximum(
        (a0p_ref[7:8] + a1p_ref[7:8] + zp_ref[7:8]) * dp_ref[7:8, 0:1] + b,
        0.0)
    hn = jnp.maximum(
        (a0n_ref[0:1] + a1n_ref[0:1] + zn_ref[0:1]) * dn8_ref[0:1, 0:1] + b,
        0.0)
    wfc = wfc_ref[...]
    dn = (((0,), (0,)), ((), ()))
    u0 = lax.dot_general(wt_ref[0], wfc, dn, precision=_PREC,
                         preferred_element_type=F32)
    u1 = lax.dot_general(wt_ref[1], wfc, dn, precision=_PREC,
                         preferred_element_type=F32)
    u2 = lax.dot_general(wt_ref[2], wfc, dn, precision=_PREC,
                         preferred_element_type=F32)
    cst = jnp.dot(btr_ref[...], wfc, preferred_element_type=F32,
                  precision=_PREC) + bfc_ref[...]
    zrow = jnp.zeros((1, D), F32)
    first = jnp.where(i > 0, hp, zrow)
    lastr = jnp.where(i < _FGRID - 1, hn, zrow)
    prev = jnp.concatenate([first, hc[:-1]], axis=0)
    nxt = jnp.concatenate([hc[1:], lastr], axis=0)
    out = jnp.dot(prev, u0, preferred_element_type=F32, precision=_PREC)
    out += jnp.dot(hc, u1, preferred_element_type=F32, precision=_PREC)
    out += jnp.dot(nxt, u2, preferred_element_type=F32, precision=_PREC)
    out_ref[...] = out + cst


def _tc_final(aggp, z, dinv8, b2r, wt_t, btr, Wfc, bfcr):
    aggf = aggp.reshape(NC * NPAD, D)
    h8 = _FBLK // 8
    pmap = lambda i: (jnp.maximum(i * h8 - 1, 0), 0)
    nmap = lambda i: (jnp.minimum((i + 1) * h8, N // 8 - 1), 0)
    pmap1 = lambda i: (NPAD // 8 + jnp.maximum(i * h8 - 1, 0), 0)
    nmap1 = lambda i: (NPAD // 8 + jnp.minimum((i + 1) * h8, N // 8 - 1), 0)
    return pl.pallas_call(
        _fin_body,
        grid=(_FGRID,),
        in_specs=[
            pl.BlockSpec((8, D), pmap),
            pl.BlockSpec((8, D), pmap1),
            pl.BlockSpec((8, D), pmap),
            pl.BlockSpec((8, 8), pmap),
            pl.BlockSpec((NC, _FBLK, D), lambda i: (0, i, 0)),
            pl.BlockSpec((_FBLK, D), lambda i: (i, 0)),
            pl.BlockSpec((_FBLK, 8), lambda i: (i, 0)),
            pl.BlockSpec((8, D), nmap),
            pl.BlockSpec((8, D), nmap1),
            pl.BlockSpec((8, D), nmap),
            pl.BlockSpec((8, 8), nmap),
            pl.BlockSpec((1, D), lambda i: (0, 0)),
            pl.BlockSpec((3, D, D), lambda i: (0, 0, 0)),
            pl.BlockSpec((1, D), lambda i: (0, 0)),
            pl.BlockSpec((D, D), lambda i: (0, 0)),
            pl.BlockSpec((1, D), lambda i: (0, 0)),
        ],
        out_specs=pl.BlockSpec((_FBLK, D), lambda i: (i, 0)),
        out_shape=jax.ShapeDtypeStruct((N, D), F32),
    )(aggf, aggf, z, dinv8,
      aggp, z, dinv8,
      aggf, aggf, z, dinv8,
      b2r, wt_t, btr, Wfc, bfcr)


def kernel(x, edge_index, W1, b1, W2, b2, Wt, bt, Wfc, bfc):
    src3 = edge_index[0].reshape(NC, NS, NCHUNK * CH)
    dst4 = edge_index[1].reshape(NC, NS, NCHUNK, CH)
    dst_pad = jnp.concatenate(
        [edge_index[1].reshape(NC * NS, E // (NC * NS)),
         jnp.full((NC * NS, NCHUNK_D * CH_D - E // (NC * NS)), NPAD - 1,
                  jnp.int32)], axis=1).reshape(NC, NS, NCHUNK_D, CH_D)
    degp = _sc_deg(dst_pad)
    xw1 = _tc_xw(x, W1)
    dinv, z1 = _tc_prep(degp, xw1)
    agg1 = _sc_scatter(z1, src3, dst4)
    z2 = _tc_mid(agg1, z1, dinv, b1.reshape(1, D), W2)
    agg2 = _sc_scatter(z2, src3, dst4)
    return _tc_final(agg2, z2, dinv, b2.reshape(1, D),
                     jnp.transpose(Wt, (2, 0, 1)), bt.reshape(1, D),
                     Wfc, bfc.reshape(1, D))
